# baseline (speedup 1.0000x reference)
import functools

import jax
import jax.numpy as jnp
from jax import lax
from jax.experimental import pallas as pl
from jax.experimental.pallas import tpu as pltpu
from jax.experimental.pallas import tpu_sc as plsc

B = 16384
D = 128
H1 = 256
H2 = 512

NC = 2
NS = 16
NW = NC * NS

NCHUNK = 2
CB = B // NCHUNK
BW_C = CB // NW
CH = 128
K_C = BW_C // CH

BM = 2048

_sc_mesh = plsc.VectorSubcoreMesh(core_axis_name="c", subcore_axis_name="s")


def _sc_gather_body(table_hbm, idx_hbm, out_hbm, idx_v, rows_v, sem):
    wid = lax.axis_index("s") * NC + lax.axis_index("c")
    pltpu.sync_copy(idx_hbm.at[wid], idx_v)
    copies = [
        pltpu.async_copy(table_hbm.at[idx_v.at[j]], rows_v.at[j], sem)
        for j in range(K_C)
    ]
    for j in range(K_C):
        copies[j].wait()
        pltpu.sync_copy(rows_v.at[j], out_hbm.at[pl.ds(wid * BW_C + j * CH, CH)])


_sc_gather = pl.kernel(
    _sc_gather_body,
    out_type=jax.ShapeDtypeStruct((CB, D), jnp.float32),
    mesh=_sc_mesh,
    scratch_types=[
        pltpu.VMEM((K_C, CH), jnp.int32),
        pltpu.VMEM((K_C, CH, D), jnp.float32),
        pltpu.SemaphoreType.DMA,
    ],
)


def _mlp_compute(emb_ref, w1_ref, b1_ref, w2_ref, b2_ref, out_ref):
    dn = (((1,), (1,)), ((), ()))
    h = lax.dot_general(emb_ref[...], w1_ref[...], dn,
                        preferred_element_type=jnp.float32)
    h = h + b1_ref[...]
    h = h * jax.nn.sigmoid(h)
    out_ref[...] = lax.dot_general(h, w2_ref[...], dn,
                                   preferred_element_type=jnp.float32) + b2_ref[...]


def _mlp_compute_alias(emb_ref, w1_ref, b1_ref, w2_ref, b2_ref, prev_ref, out_ref):
    del prev_ref
    _mlp_compute(emb_ref, w1_ref, b1_ref, w2_ref, b2_ref, out_ref)


def _mlp_chunk(c, emb_c, w1, b1, w2, b2, prev):
    nb = CB // BM
    off = c * nb
    in_specs = [
        pl.BlockSpec((BM, D), lambda i: (i, 0)),
        pl.BlockSpec((H1, D), lambda i: (0, 0)),
        pl.BlockSpec((H1,), lambda i: (0,)),
        pl.BlockSpec((H2, H1), lambda i: (0, 0)),
        pl.BlockSpec((H2,), lambda i: (0,)),
    ]
    args = [emb_c, w1, b1, w2, b2]
    if prev is None:
        body = _mlp_compute
        kwargs = {}
    else:
        body = _mlp_compute_alias
        in_specs.append(pl.BlockSpec(memory_space=pl.ANY))
        args.append(prev)
        kwargs = dict(input_output_aliases={5: 0})
    return pl.pallas_call(
        body,
        grid=(nb,),
        in_specs=in_specs,
        out_specs=pl.BlockSpec((BM, H2), lambda i: (off + i, 0)),
        out_shape=jax.ShapeDtypeStruct((B, H2), jnp.float32),
        **kwargs,
    )(*args)


def kernel(class_labels, emb_table, W1, b1, W2, b2):
    idx = class_labels.astype(jnp.int32).reshape(NCHUNK, NW, K_C, CH)
    embs = [_sc_gather(emb_table, idx[c]) for c in range(NCHUNK)]
    out = None
    for c in range(NCHUNK):
        out = _mlp_chunk(c, embs[c], W1, b1, W2, b2, out)
    return out


# speedup vs baseline: 1.0145x; 1.0145x over previous
import functools

import jax
import jax.numpy as jnp
from jax import lax
from jax.experimental import pallas as pl
from jax.experimental.pallas import tpu as pltpu
from jax.experimental.pallas import tpu_sc as plsc

B = 16384
D = 128
H1 = 256
H2 = 512

NC = 2
NS = 16
NW = NC * NS

NCHUNK = 2
CB = B // NCHUNK
BW_C = CB // NW
CH = 128
K_C = BW_C // CH

BM = 2048

_sc_mesh = plsc.VectorSubcoreMesh(core_axis_name="c", subcore_axis_name="s")


def _sc_gather_body(table_hbm, idx_hbm, out_hbm, idx_v, rows_v, sem):
    wid = lax.axis_index("s") * NC + lax.axis_index("c")
    pltpu.sync_copy(idx_hbm.at[wid], idx_v)
    copies = [
        pltpu.async_copy(table_hbm.at[idx_v.at[j]], rows_v.at[j], sem)
        for j in range(K_C)
    ]
    for j in range(K_C):
        copies[j].wait()
        pltpu.sync_copy(rows_v.at[j], out_hbm.at[pl.ds(wid * BW_C + j * CH, CH)])


_sc_gather = pl.kernel(
    _sc_gather_body,
    out_type=jax.ShapeDtypeStruct((CB, D), jnp.float32),
    mesh=_sc_mesh,
    scratch_types=[
        pltpu.VMEM((K_C, CH), jnp.int32),
        pltpu.VMEM((K_C, CH, D), jnp.float32),
        pltpu.SemaphoreType.DMA,
    ],
)


NB = CB // BM
NSLOT = 2


def _mlp_compute(emb_ref, w1_ref, b1_ref, w2_ref, b2_ref, out_ref):
    dn = (((1,), (1,)), ((), ()))
    h = lax.dot_general(emb_ref[...], w1_ref[...], dn,
                        preferred_element_type=jnp.float32)
    h = h + b1_ref[...]
    h = h * jax.nn.sigmoid(h)
    out_ref[...] = lax.dot_general(h, w2_ref[...], dn,
                                   preferred_element_type=jnp.float32) + b2_ref[...]


def _mlp_body(c, emb_ref, w1_ref, b1_ref, w2_ref, b2_ref, out_hbm, otile, sem):
    i = pl.program_id(0)
    row = (c * NB + i) * BM

    def copy(slot):
        return pltpu.make_async_copy(
            otile.at[slot], out_hbm.at[pl.ds(row, BM)], sem.at[slot])

    slot = lax.rem(i, NSLOT)

    @pl.when(i >= NSLOT)
    def _():
        copy(slot).wait()

    _mlp_compute(emb_ref, w1_ref, b1_ref, w2_ref, b2_ref, otile.at[slot])
    copy(slot).start()

    @pl.when(i == NB - 1)
    def _():
        for s in range(NSLOT):
            pltpu.make_async_copy(
                otile.at[s], out_hbm.at[pl.ds(row, BM)], sem.at[s]).wait()


def _mlp_body_alias(c, emb_ref, w1_ref, b1_ref, w2_ref, b2_ref, prev_ref,
                    out_hbm, otile, sem):
    del prev_ref
    _mlp_body(c, emb_ref, w1_ref, b1_ref, w2_ref, b2_ref, out_hbm, otile, sem)


def _mlp_chunk(c, emb_c, w1, b1, w2, b2, prev):
    in_specs = [
        pl.BlockSpec((BM, D), lambda i: (i, 0)),
        pl.BlockSpec((H1, D), lambda i: (0, 0)),
        pl.BlockSpec((H1,), lambda i: (0,)),
        pl.BlockSpec((H2, H1), lambda i: (0, 0)),
        pl.BlockSpec((H2,), lambda i: (0,)),
    ]
    args = [emb_c, w1, b1, w2, b2]
    if prev is None:
        body = functools.partial(_mlp_body, c)
        kwargs = {}
    else:
        body = functools.partial(_mlp_body_alias, c)
        in_specs.append(pl.BlockSpec(memory_space=pl.ANY))
        args.append(prev)
        kwargs = dict(input_output_aliases={5: 0})
    return pl.pallas_call(
        body,
        grid=(NB,),
        in_specs=in_specs,
        out_specs=pl.BlockSpec(memory_space=pl.ANY),
        out_shape=jax.ShapeDtypeStruct((B, H2), jnp.float32),
        scratch_shapes=[
            pltpu.VMEM((NSLOT, BM, H2), jnp.float32),
            pltpu.SemaphoreType.DMA((NSLOT,)),
        ],
        **kwargs,
    )(*args)


def kernel(class_labels, emb_table, W1, b1, W2, b2):
    idx = class_labels.astype(jnp.int32).reshape(NCHUNK, NW, K_C, CH)
    embs = [_sc_gather(emb_table, idx[c]) for c in range(NCHUNK)]
    out = None
    for c in range(NCHUNK):
        out = _mlp_chunk(c, embs[c], W1, b1, W2, b2, out)
    return out


# speedup vs baseline: 1.0804x; 1.0650x over previous
import functools

import jax
import jax.numpy as jnp
from jax import lax
from jax.experimental import pallas as pl
from jax.experimental.pallas import tpu as pltpu
from jax.experimental.pallas import tpu_sc as plsc

B = 16384
D = 128
H1 = 256
H2 = 512

NC = 2
NS = 16
NW = NC * NS

NCHUNK = 1
CB = B // NCHUNK
BW_C = CB // NW
CH = 128
K_C = BW_C // CH

BM = 4096

_sc_mesh = plsc.VectorSubcoreMesh(core_axis_name="c", subcore_axis_name="s")


def _sc_gather_body(table_hbm, idx_hbm, out_hbm, idx_v, rows_v, sem):
    wid = lax.axis_index("s") * NC + lax.axis_index("c")
    pltpu.sync_copy(idx_hbm.at[wid], idx_v)
    copies = [
        pltpu.async_copy(table_hbm.at[idx_v.at[j]], rows_v.at[j], sem)
        for j in range(K_C)
    ]
    for j in range(K_C):
        copies[j].wait()
        pltpu.sync_copy(rows_v.at[j], out_hbm.at[pl.ds(wid * BW_C + j * CH, CH)])


_sc_gather = pl.kernel(
    _sc_gather_body,
    out_type=jax.ShapeDtypeStruct((CB, D), jnp.float32),
    mesh=_sc_mesh,
    scratch_types=[
        pltpu.VMEM((K_C, CH), jnp.int32),
        pltpu.VMEM((K_C, CH, D), jnp.float32),
        pltpu.SemaphoreType.DMA,
    ],
)


NB = CB // BM
NSLOT = 2


def _mlp_compute(emb_ref, w1_ref, b1_ref, w2_ref, b2_ref, out_ref):
    dn = (((1,), (1,)), ((), ()))
    h = lax.dot_general(emb_ref[...], w1_ref[...], dn,
                        preferred_element_type=jnp.float32)
    h = h + b1_ref[...]
    h = h * jax.nn.sigmoid(h)
    out_ref[...] = lax.dot_general(h, w2_ref[...], dn,
                                   preferred_element_type=jnp.float32) + b2_ref[...]


def _mlp_body(c, emb_ref, w1_ref, b1_ref, w2_ref, b2_ref, out_hbm, otile, sem):
    i = pl.program_id(0)
    row = (c * NB + i) * BM

    def copy(slot):
        return pltpu.make_async_copy(
            otile.at[slot], out_hbm.at[pl.ds(row, BM)], sem.at[slot])

    slot = lax.rem(i, NSLOT)

    @pl.when(i >= NSLOT)
    def _():
        copy(slot).wait()

    _mlp_compute(emb_ref, w1_ref, b1_ref, w2_ref, b2_ref, otile.at[slot])
    copy(slot).start()

    @pl.when(i == NB - 1)
    def _():
        for s in range(NSLOT):
            pltpu.make_async_copy(
                otile.at[s], out_hbm.at[pl.ds(row, BM)], sem.at[s]).wait()


def _mlp_body_alias(c, emb_ref, w1_ref, b1_ref, w2_ref, b2_ref, prev_ref,
                    out_hbm, otile, sem):
    del prev_ref
    _mlp_body(c, emb_ref, w1_ref, b1_ref, w2_ref, b2_ref, out_hbm, otile, sem)


def _mlp_chunk(c, emb_c, w1, b1, w2, b2, prev):
    in_specs = [
        pl.BlockSpec((BM, D), lambda i: (i, 0)),
        pl.BlockSpec((H1, D), lambda i: (0, 0)),
        pl.BlockSpec((H1,), lambda i: (0,)),
        pl.BlockSpec((H2, H1), lambda i: (0, 0)),
        pl.BlockSpec((H2,), lambda i: (0,)),
    ]
    args = [emb_c, w1, b1, w2, b2]
    if prev is None:
        body = functools.partial(_mlp_body, c)
        kwargs = {}
    else:
        body = functools.partial(_mlp_body_alias, c)
        in_specs.append(pl.BlockSpec(memory_space=pl.ANY))
        args.append(prev)
        kwargs = dict(input_output_aliases={5: 0})
    return pl.pallas_call(
        body,
        grid=(NB,),
        in_specs=in_specs,
        out_specs=pl.BlockSpec(memory_space=pl.ANY),
        out_shape=jax.ShapeDtypeStruct((B, H2), jnp.float32),
        scratch_shapes=[
            pltpu.VMEM((NSLOT, BM, H2), jnp.float32),
            pltpu.SemaphoreType.DMA((NSLOT,)),
        ],
        **kwargs,
    )(*args)


def kernel(class_labels, emb_table, W1, b1, W2, b2):
    idx = class_labels.astype(jnp.int32).reshape(NCHUNK, NW, K_C, CH)
    embs = [_sc_gather(emb_table, idx[c]) for c in range(NCHUNK)]
    out = None
    for c in range(NCHUNK):
        out = _mlp_chunk(c, embs[c], W1, b1, W2, b2, out)
    return out
